# baseline (speedup 1.0000x reference)
import functools

import jax
import jax.numpy as jnp
from jax import lax
from jax.experimental import pallas as pl
from jax.experimental.pallas import tpu as pltpu
from jax.experimental.pallas import tpu_sc as plsc

N_P, N_L, N_ALL, N_E = 8000, 2000, 10000, 160000
B, HID, PF, LF, T = 16, 128, 27, 13, 1000

W144 = HID + 16
N_TAB = 10240
PAD_IDX = 10112
PAD_LOCAL = PAD_IDX - N_P
NC, NS, NW = 2, 16, 32
CHUNK = 128
NCH = 40
E_PAD = NW * NCH * CHUNK
E_PER_TILE = NCH * CHUNK
G16 = E_PER_TILE // 16
ACC_ROWS = 2304
LIG_ROWS = N_TAB - N_P
ROWS_PER_TILE = LIG_ROWS // NS
EBLK = 512
NCHB = E_PER_TILE // EBLK
GCH = 32
QUAD = 8
QROWS = GCH * QUAD


def _node_tables_kernel(ppos_ref, pv_ref, bp_ref, lpos_ref, lv_ref, bl_ref,
                        tf_ref, Wp_ref, bpv_ref, Wl_ref, blv_ref,
                        We1a_ref, We1c_ref, be1_ref,
                        ta_ref, tc_ref, tx_ref, hlig_ref):
    bp = bp_ref[...]
    ohp = (bp == lax.broadcasted_iota(jnp.int32, (N_P, B), 1)).astype(jnp.float32)
    ppos = ppos_ref[...]
    seg = lax.dot_general(ohp, ppos, (((0,), (0,)), ((), ())))
    cnt = lax.dot_general(ohp, jnp.ones((N_P, 1), jnp.float32),
                          (((0,), (0,)), ((), ())))
    offset = seg / jnp.maximum(cnt, 1.0)
    bl = bl_ref[...]
    ohl = (bl == lax.broadcasted_iota(jnp.int32, (N_L, B), 1)).astype(jnp.float32)
    pc = ppos - ohp @ offset
    lc = lpos_ref[...] - ohl @ offset

    h_p = pv_ref[...] @ Wp_ref[...] + bpv_ref[...]
    onehot = (lv_ref[...] == lax.broadcasted_iota(jnp.int32, (N_L, LF), 1)
              ).astype(jnp.float32)
    tfeat = ohl @ tf_ref[...]
    lig_feat = jnp.concatenate([onehot, tfeat], axis=1)
    h_l = lig_feat @ Wl_ref[...] + blv_ref[...]

    We1a = We1a_ref[...]
    We1c = We1c_ref[...]
    a_p = h_p @ We1a[:127] + be1_ref[...]
    a_l = h_l @ We1a[:127] + We1a[127:128] + be1_ref[...]
    c_l = h_l @ We1c[:127] + We1c[127:128]

    px16 = jnp.concatenate([pc, jnp.zeros((N_P, 13), jnp.float32)], axis=1)
    lx16 = jnp.concatenate([lc, jnp.zeros((N_L, 13), jnp.float32)], axis=1)
    ta_ref[0:N_P, :] = jnp.concatenate([a_p, px16], axis=1)
    ta_ref[N_P:N_ALL, :] = jnp.concatenate([a_l, lx16], axis=1)
    ta_ref[N_ALL:N_TAB, :] = jnp.zeros((N_TAB - N_ALL, W144), jnp.float32)
    tc_ref[0:N_L, :] = jnp.concatenate([c_l, lx16], axis=1)
    tc_ref[N_L:ACC_ROWS, :] = jnp.zeros((ACC_ROWS - N_L, W144), jnp.float32)
    tx_ref[...] = lx16
    hlig_ref[...] = jnp.concatenate(
        [h_l, jnp.ones((N_L, 1), jnp.float32)], axis=1)


def _sc_gather_kernel(ta, tc_t, src_i, dst_i, slotv, counts,
                      a_out, c_out, dstc_out,
                      sraw, draw, slotbuf, scmp, dcmp,
                      abig, cbig, cnt_v, sem):
    wid = lax.axis_index("s") * NC + lax.axis_index("c")
    base = wid * E_PER_TILE
    pltpu.sync_copy(src_i.at[pl.ds(base, E_PER_TILE)], sraw)
    pltpu.sync_copy(dst_i.at[pl.ds(base, E_PER_TILE)], draw)
    pltpu.sync_copy(slotv.at[pl.ds(base, E_PER_TILE)], slotbuf)
    pltpu.sync_copy(counts.at[wid], cnt_v)
    cnt = jnp.max(cnt_v[...])

    def compact(i, carry):
        dv = draw[pl.ds(i * 16, 16)]
        sv = sraw[pl.ds(i * 16, 16)]
        pv = slotbuf[pl.ds(i * 16, 16)]
        dl = dv - N_P
        neg = lax.shift_right_logical(dl, 31)
        dval = dl + neg * (PAD_LOCAL - dl)
        sval = sv + neg * (PAD_IDX - sv)
        plsc.store_scatter(dcmp, [pv], dval)
        plsc.store_scatter(scmp, [pv], sval)
        return carry

    lax.fori_loop(0, G16, compact, 0)
    pltpu.sync_copy(dcmp.at[pl.ds(0, E_PER_TILE)],
                    dstc_out.at[pl.ds(base, E_PER_TILE)])

    def quad(iq, carry):
        @pl.when(iq * QROWS < cnt)
        def _():
            descs = []
            for q in range(QUAD):
                off = iq * QROWS + q * GCH
                d_a = pltpu.async_copy(
                    ta.at[scmp.at[pl.ds(off, GCH)]],
                    abig.at[pl.ds(q * GCH, GCH)], sem)
                d_c = pltpu.async_copy(
                    tc_t.at[dcmp.at[pl.ds(off, GCH)]],
                    cbig.at[pl.ds(q * GCH, GCH)], sem)
                descs.append(d_a)
                descs.append(d_c)
            for d in descs:
                d.wait()
            qbase = base + iq * QROWS
            pltpu.sync_copy(abig, a_out.at[pl.ds(qbase, QROWS)])
            pltpu.sync_copy(cbig, c_out.at[pl.ds(qbase, QROWS)])
        return carry

    lax.fori_loop(0, E_PER_TILE // QROWS, quad, 0)


def _edge_mlp_kernel(counts_ref, a_ref, c_ref,
                     w1l_ref, We2_ref, be2_ref, Wx1_ref, bx1_ref,
                     wx2_ref, bx2_ref, y_ref):
    t = pl.program_id(0)
    j = pl.program_id(1)

    @pl.when(j * EBLK < counts_ref[t])
    def _():
        a = a_ref[...]
        c = c_ref[...]
        pre = a[:, 0:HID] + c[:, 0:HID]
        rel16 = c[:, HID:W144] - a[:, HID:W144]
        d2 = jnp.sum(rel16 * rel16, axis=1, keepdims=True)
        m1 = jax.nn.silu(pre + d2 * w1l_ref[...])
        m2 = jax.nn.silu(m1 @ We2_ref[...] + be2_ref[...])
        u = jax.nn.silu(m2 @ Wx1_ref[...] + bx1_ref[...])
        xw = jnp.sum(u * wx2_ref[...], axis=1, keepdims=True) + bx2_ref[0, 0]
        y_ref[...] = jnp.concatenate([m2, rel16 * jnp.tanh(xw)], axis=1)


def _sc_scatter_kernel(ym, dstc, counts, agg_out,
                       idx_v, mbuf, zm, cnt_v, agg_acc):
    c = lax.axis_index("c")
    s = lax.axis_index("s")
    wid = s * NC + c
    pltpu.sync_copy(counts.at[wid], cnt_v)
    cnt = jnp.max(cnt_v[...])

    def zrow(i, carry):
        for g in range(W144 // 16):
            zm[i, pl.ds(g * 16, 16)] = jnp.zeros((16,), jnp.float32)
        return carry

    lax.fori_loop(0, CHUNK, zrow, 0)
    zbase = s * (ACC_ROWS // NS)
    pltpu.sync_copy(zm, agg_acc.at[pl.ds(zbase, CHUNK)])
    pltpu.sync_copy(zm.at[pl.ds(0, 16)], agg_acc.at[pl.ds(zbase + CHUNK, 16)])
    plsc.subcore_barrier()

    def body(i, carry):
        @pl.when(i * CHUNK < cnt)
        def _():
            base = wid * E_PER_TILE + i * CHUNK
            pltpu.sync_copy(dstc.at[pl.ds(base, CHUNK)], idx_v)
            pltpu.sync_copy(ym.at[pl.ds(base, CHUNK)], mbuf)
            pltpu.sync_copy(mbuf, agg_acc.at[idx_v], add=True)
        return carry

    lax.fori_loop(0, NCH, body, 0)
    plsc.subcore_barrier()
    rbase = s * ROWS_PER_TILE
    pltpu.sync_copy(agg_acc.at[pl.ds(rbase, ROWS_PER_TILE)],
                    agg_out.at[c, pl.ds(rbase, ROWS_PER_TILE)])


def _final_kernel(hlig_ref, tx_ref, pm_ref,
                  Wn1_ref, bn1_ref, Wn2_ref, bn2_ref, Wv_ref, bv_ref,
                  out_ref):
    h = hlig_ref[...]
    p = pm_ref[0, 0:N_L, :] + pm_ref[1, 0:N_L, :]
    agg = p[:, 0:HID]
    dx16 = p[:, HID:W144]
    x16 = tx_ref[...]
    cat = jnp.concatenate([h, agg], axis=1)
    hn = jax.nn.silu(cat @ Wn1_ref[...] + bn1_ref[...])
    hnew = h + hn @ Wn2_ref[...] + bn2_ref[...]
    out_ref[...] = x16 + dx16 + hnew @ Wv_ref[...] + bv_ref[...]


def kernel(protein_pos, protein_v, batch_protein, init_ligand_pos,
           init_ligand_v, batch_ligand, time_step, edge_index,
           Wp, bp, Wl, bl, We1, be1, We2, be2, Wn1, bn1, Wn2, bn2,
           Wx1, bx1, Wx2, bx2, Wv, bv):
    f32 = jnp.float32

    bp2 = batch_protein.astype(jnp.int32).reshape(N_P, 1)
    bl2 = batch_ligand.astype(jnp.int32).reshape(N_L, 1)
    lv2 = init_ligand_v.astype(jnp.int32).reshape(N_L, 1)
    tf = (time_step.astype(f32) / T).reshape(B, 1)
    We1a = We1[0:HID]
    We1c = We1[HID:2 * HID]
    w1last = We1[2 * HID].reshape(1, HID)
    pad = jnp.full((E_PAD - N_E,), PAD_IDX, jnp.int32)
    src_i = jnp.concatenate([edge_index[0].astype(jnp.int32), pad])
    dst_i = jnp.concatenate([edge_index[1].astype(jnp.int32), pad])
    keep = (dst_i >= N_P).astype(jnp.int32).reshape(NW, E_PER_TILE)
    pref_k = jnp.cumsum(keep, axis=1) - keep
    drop = 1 - keep
    pref_d = jnp.cumsum(drop, axis=1) - drop
    counts32 = keep.sum(axis=1)
    slotv = jnp.where(keep == 1, pref_k,
                      counts32[:, None] + pref_d).reshape(E_PAD)
    counts2d = jnp.broadcast_to(counts32[:, None], (NW, 16))
    Wv16 = jnp.concatenate([jnp.zeros((HID, 3), f32), Wv], axis=1)
    bv16 = jnp.concatenate([jnp.zeros((3,), f32), bv]).reshape(1, 16)

    ta, tc_t, tx, hlig = pl.pallas_call(
        _node_tables_kernel,
        out_shape=[
            jax.ShapeDtypeStruct((N_TAB, W144), f32),
            jax.ShapeDtypeStruct((ACC_ROWS, W144), f32),
            jax.ShapeDtypeStruct((N_L, 16), f32),
            jax.ShapeDtypeStruct((N_L, HID), f32),
        ],
    )(protein_pos, protein_v, bp2, init_ligand_pos, lv2, bl2, tf,
      Wp, bp.reshape(1, HID - 1), Wl, bl.reshape(1, HID - 1),
      We1a, We1c, be1.reshape(1, HID))

    mesh = plsc.VectorSubcoreMesh(core_axis_name="c", subcore_axis_name="s")
    sc_params = pltpu.CompilerParams(use_tc_tiling_on_sc=False,
                                     needs_layout_passes=False)
    gather = functools.partial(
        pl.kernel, _sc_gather_kernel, mesh=mesh, compiler_params=sc_params,
        out_type=[
            jax.ShapeDtypeStruct((E_PAD, W144), f32),
            jax.ShapeDtypeStruct((E_PAD, W144), f32),
            jax.ShapeDtypeStruct((E_PAD,), jnp.int32),
        ],
        scratch_types=[
            pltpu.VMEM((E_PER_TILE,), jnp.int32),
            pltpu.VMEM((E_PER_TILE,), jnp.int32),
            pltpu.VMEM((E_PER_TILE,), jnp.int32),
            pltpu.VMEM((E_PER_TILE,), jnp.int32),
            pltpu.VMEM((E_PER_TILE,), jnp.int32),
            pltpu.VMEM((QROWS, W144), f32),
            pltpu.VMEM((QROWS, W144), f32),
            pltpu.VMEM((16,), jnp.int32),
            pltpu.SemaphoreType.DMA,
        ],
    )()
    a_rows, c_rows, dstc = gather(ta, tc_t, src_i, dst_i, slotv, counts2d)

    def edge_blk(r, cdim):
        def im(t, j, counts):
            jm = jnp.maximum((counts[t] + EBLK - 1) // EBLK - 1, 0)
            return (t * NCHB + jnp.minimum(j, jm), 0)
        return pl.BlockSpec((r, cdim), im)

    rep = lambda r, cdim: pl.BlockSpec((r, cdim), lambda t, j, counts: (0, 0))
    y_rows = pl.pallas_call(
        _edge_mlp_kernel,
        grid_spec=pltpu.PrefetchScalarGridSpec(
            num_scalar_prefetch=1,
            grid=(NW, NCHB),
            in_specs=[
                edge_blk(EBLK, W144), edge_blk(EBLK, W144),
                rep(1, HID), rep(HID, HID), rep(1, HID), rep(HID, HID),
                rep(1, HID), rep(1, HID), rep(1, 1),
            ],
            out_specs=[edge_blk(EBLK, W144)],
        ),
        out_shape=[jax.ShapeDtypeStruct((E_PAD, W144), f32)],
    )(counts32, a_rows, c_rows, w1last, We2,
      be2.reshape(1, HID), Wx1, bx1.reshape(1, HID),
      Wx2.reshape(1, HID), bx2.reshape(1, 1))[0]

    scatter = functools.partial(
        pl.kernel, _sc_scatter_kernel, mesh=mesh, compiler_params=sc_params,
        out_type=[jax.ShapeDtypeStruct((NC, LIG_ROWS, W144), f32)],
        scratch_types=[
            pltpu.VMEM((CHUNK,), jnp.int32),
            pltpu.VMEM((CHUNK, W144), f32),
            pltpu.VMEM((CHUNK, W144), f32),
            pltpu.VMEM((16,), jnp.int32),
            pltpu.VMEM_SHARED((ACC_ROWS, W144), f32),
        ],
    )()
    pm = scatter(y_rows, dstc, counts2d)[0]

    out = pl.pallas_call(
        _final_kernel,
        out_shape=jax.ShapeDtypeStruct((N_L, 16), f32),
    )(hlig, tx, pm, Wn1, bn1.reshape(1, HID), Wn2, bn2.reshape(1, HID),
      Wv16, bv16)
    return out


# speedup vs baseline: 1.0974x; 1.0974x over previous
import functools

import jax
import jax.numpy as jnp
from jax import lax
from jax.experimental import pallas as pl
from jax.experimental.pallas import tpu as pltpu
from jax.experimental.pallas import tpu_sc as plsc

N_P, N_L, N_ALL, N_E = 8000, 2000, 10000, 160000
B, HID, PF, LF, T = 16, 128, 27, 13, 1000

N_TAB = 10240
PAD_IDX = 10112
PAD_LOCAL = PAD_IDX - N_P
NC, NS, NW = 2, 16, 32
CHUNK = 128
NCH = 40
E_PAD = NW * NCH * CHUNK
E_PER_TILE = NCH * CHUNK
G16 = E_PER_TILE // 16
CBUF = E_PER_TILE
ACC_ROWS = 2304
LIG_ROWS = N_TAB - N_P
ROWS_PER_TILE = LIG_ROWS // NS
EBLK = 512
NCHB = E_PER_TILE // EBLK


def _node_tables_kernel(ppos_ref, pv_ref, bp_ref, lpos_ref, lv_ref, bl_ref,
                        tf_ref, Wp_ref, bpv_ref, Wl_ref, blv_ref,
                        We1a_ref, We1c_ref, be1_ref,
                        ta_ref, tc_ref, tx_ref, hlig_ref):
    bp = bp_ref[...]
    ohp = (bp == lax.broadcasted_iota(jnp.int32, (N_P, B), 1)).astype(jnp.float32)
    ppos = ppos_ref[...]
    seg = lax.dot_general(ohp, ppos, (((0,), (0,)), ((), ())))
    cnt = lax.dot_general(ohp, jnp.ones((N_P, 1), jnp.float32),
                          (((0,), (0,)), ((), ())))
    offset = seg / jnp.maximum(cnt, 1.0)
    bl = bl_ref[...]
    ohl = (bl == lax.broadcasted_iota(jnp.int32, (N_L, B), 1)).astype(jnp.float32)
    pc = ppos - ohp @ offset
    lc = lpos_ref[...] - ohl @ offset

    h_p = pv_ref[...] @ Wp_ref[...] + bpv_ref[...]
    onehot = (lv_ref[...] == lax.broadcasted_iota(jnp.int32, (N_L, LF), 1)
              ).astype(jnp.float32)
    tfeat = ohl @ tf_ref[...]
    lig_feat = jnp.concatenate([onehot, tfeat], axis=1)
    h_l = lig_feat @ Wl_ref[...] + blv_ref[...]

    We1a = We1a_ref[...]
    We1c = We1c_ref[...]
    a_p = h_p @ We1a[:127] + be1_ref[...]
    c_p = h_p @ We1c[:127]
    a_l = h_l @ We1a[:127] + We1a[127:128] + be1_ref[...]
    c_l = h_l @ We1c[:127] + We1c[127:128]

    ta_ref[0:N_P, :] = a_p
    ta_ref[N_P:N_ALL, :] = a_l
    ta_ref[N_ALL:N_TAB, :] = jnp.zeros((N_TAB - N_ALL, HID), jnp.float32)
    tc_ref[0:N_P, :] = c_p
    tc_ref[N_P:N_ALL, :] = c_l
    tc_ref[N_ALL:N_TAB, :] = jnp.zeros((N_TAB - N_ALL, HID), jnp.float32)
    tx_ref[0:N_P, :] = jnp.concatenate(
        [pc, jnp.zeros((N_P, 125), jnp.float32)], axis=1)
    tx_ref[N_P:N_ALL, :] = jnp.concatenate(
        [lc, jnp.zeros((N_L, 125), jnp.float32)], axis=1)
    tx_ref[N_ALL:N_TAB, :] = jnp.zeros((N_TAB - N_ALL, HID), jnp.float32)
    hlig_ref[...] = jnp.concatenate(
        [h_l, jnp.ones((N_L, 1), jnp.float32)], axis=1)


def _sc_gather_kernel(ta, tc_t, tx, src_i, dst_i, slotv, counts,
                      a_out, c_out, xs_out, xd_out, dstc_out,
                      sraw, draw, slotbuf, scmp, dcmp, idxs_v, idxd_v,
                      abuf, cbuf, xsbuf, xdbuf, cnt_v, sem):
    wid = lax.axis_index("s") * NC + lax.axis_index("c")
    base = wid * E_PER_TILE
    pltpu.sync_copy(src_i.at[pl.ds(base, E_PER_TILE)], sraw)
    pltpu.sync_copy(dst_i.at[pl.ds(base, E_PER_TILE)], draw)
    pltpu.sync_copy(slotv.at[pl.ds(base, E_PER_TILE)], slotbuf)
    pltpu.sync_copy(counts.at[wid], cnt_v)
    cnt = jnp.max(cnt_v[...])

    def compact(i, carry):
        dv = draw[pl.ds(i * 16, 16)]
        sv = sraw[pl.ds(i * 16, 16)]
        pv = slotbuf[pl.ds(i * 16, 16)]
        dl = dv - N_P
        neg = lax.shift_right_logical(dl, 31)
        dval = dl + neg * (PAD_LOCAL - dl)
        sval = sv + neg * (PAD_IDX - sv)
        plsc.store_scatter(dcmp, [pv], dval)
        plsc.store_scatter(scmp, [pv], sval)
        return carry

    lax.fori_loop(0, G16, compact, 0)
    pltpu.sync_copy(dcmp.at[pl.ds(0, E_PER_TILE)],
                    dstc_out.at[pl.ds(base, E_PER_TILE)])

    def body(i, carry):
        @pl.when(i * CHUNK < cnt)
        def _():
            off = i * CHUNK
            for g in range(CHUNK // 16):
                idxs_v[pl.ds(g * 16, 16)] = scmp[pl.ds(off + g * 16, 16)]
                idxd_v[pl.ds(g * 16, 16)] = dcmp[pl.ds(off + g * 16, 16)] + N_P
            d_a = pltpu.async_copy(ta.at[idxs_v], abuf, sem)
            d_c = pltpu.async_copy(tc_t.at[idxd_v], cbuf, sem)
            d_xs = pltpu.async_copy(tx.at[idxs_v], xsbuf, sem)
            d_xd = pltpu.async_copy(tx.at[idxd_v], xdbuf, sem)
            d_a.wait(); d_c.wait(); d_xs.wait(); d_xd.wait()
            pltpu.sync_copy(abuf, a_out.at[pl.ds(base + off, CHUNK)])
            pltpu.sync_copy(cbuf, c_out.at[pl.ds(base + off, CHUNK)])
            pltpu.sync_copy(xsbuf, xs_out.at[pl.ds(base + off, CHUNK)])
            pltpu.sync_copy(xdbuf, xd_out.at[pl.ds(base + off, CHUNK)])
        return carry

    lax.fori_loop(0, NCH, body, 0)


def _edge_mlp_kernel(counts_ref, a_ref, c_ref, xs_ref, xd_ref,
                     w1l_ref, We2_ref, be2_ref, Wx1_ref, bx1_ref,
                     wx2_ref, bx2_ref, m_ref, w_ref):
    t = pl.program_id(0)
    j = pl.program_id(1)

    @pl.when(j * EBLK < counts_ref[t])
    def _():
        pre = a_ref[...] + c_ref[...]
        rel16 = xd_ref[...] - xs_ref[...]
        d2 = jnp.sum(rel16 * rel16, axis=1, keepdims=True)
        m1 = jax.nn.silu(pre + d2 * w1l_ref[...])
        m2 = jax.nn.silu(m1 @ We2_ref[...] + be2_ref[...])
        u = jax.nn.silu(m2 @ Wx1_ref[...] + bx1_ref[...])
        xw = jnp.sum(u * wx2_ref[...], axis=1, keepdims=True) + bx2_ref[0, 0]
        m_ref[...] = m2
        w_ref[...] = rel16 * jnp.tanh(xw)


def _sc_scatter_kernel(ym, yw, dstc, counts, aggm_out, aggw_out,
                       idx_v, mbuf, wbuf, zm, zw, cnt_v,
                       aggm_acc, aggw_acc):
    c = lax.axis_index("c")
    s = lax.axis_index("s")
    wid = s * NC + c
    pltpu.sync_copy(counts.at[wid], cnt_v)
    cnt = jnp.max(cnt_v[...])

    def zrow(i, carry):
        for g in range(HID // 16):
            zm[i, pl.ds(g * 16, 16)] = jnp.zeros((16,), jnp.float32)
            zw[i, pl.ds(g * 16, 16)] = jnp.zeros((16,), jnp.float32)
        return carry

    lax.fori_loop(0, CHUNK, zrow, 0)
    zbase = s * (ACC_ROWS // NS)
    pltpu.sync_copy(zm, aggm_acc.at[pl.ds(zbase, CHUNK)])
    pltpu.sync_copy(zw, aggw_acc.at[pl.ds(zbase, CHUNK)])
    pltpu.sync_copy(zm.at[pl.ds(0, 16)],
                    aggm_acc.at[pl.ds(zbase + CHUNK, 16)])
    pltpu.sync_copy(zw.at[pl.ds(0, 16)],
                    aggw_acc.at[pl.ds(zbase + CHUNK, 16)])
    plsc.subcore_barrier()

    def body(i, carry):
        @pl.when(i * CHUNK < cnt)
        def _():
            base = wid * E_PER_TILE + i * CHUNK
            pltpu.sync_copy(dstc.at[pl.ds(base, CHUNK)], idx_v)
            pltpu.sync_copy(ym.at[pl.ds(base, CHUNK)], mbuf)
            pltpu.sync_copy(yw.at[pl.ds(base, CHUNK)], wbuf)
            pltpu.sync_copy(mbuf, aggm_acc.at[idx_v], add=True)
            pltpu.sync_copy(wbuf, aggw_acc.at[idx_v], add=True)
        return carry

    lax.fori_loop(0, NCH, body, 0)
    plsc.subcore_barrier()
    rbase = s * (ACC_ROWS // NS)
    pltpu.sync_copy(aggm_acc.at[pl.ds(rbase, ACC_ROWS // NS)],
                    aggm_out.at[c, pl.ds(rbase, ACC_ROWS // NS)])
    pltpu.sync_copy(aggw_acc.at[pl.ds(rbase, ACC_ROWS // NS)],
                    aggw_out.at[c, pl.ds(rbase, ACC_ROWS // NS)])


def _final_kernel(hlig_ref, tx_ref, pm_ref, pw_ref,
                  Wn1_ref, bn1_ref, Wn2_ref, bn2_ref, Wv_ref, bv_ref,
                  out_ref):
    h = hlig_ref[...]
    agg = pm_ref[0, 0:N_L, :] + pm_ref[1, 0:N_L, :]
    dx = pw_ref[0, 0:N_L, :] + pw_ref[1, 0:N_L, :]
    x16 = tx_ref[N_P:N_ALL, 0:16]
    cat = jnp.concatenate([h, agg], axis=1)
    hn = jax.nn.silu(cat @ Wn1_ref[...] + bn1_ref[...])
    hnew = h + hn @ Wn2_ref[...] + bn2_ref[...]
    out_ref[...] = x16 + dx[:, 0:16] + hnew @ Wv_ref[...] + bv_ref[...]


def kernel(protein_pos, protein_v, batch_protein, init_ligand_pos,
           init_ligand_v, batch_ligand, time_step, edge_index,
           Wp, bp, Wl, bl, We1, be1, We2, be2, Wn1, bn1, Wn2, bn2,
           Wx1, bx1, Wx2, bx2, Wv, bv):
    f32 = jnp.float32

    bp2 = batch_protein.astype(jnp.int32).reshape(N_P, 1)
    bl2 = batch_ligand.astype(jnp.int32).reshape(N_L, 1)
    lv2 = init_ligand_v.astype(jnp.int32).reshape(N_L, 1)
    tf = (time_step.astype(f32) / T).reshape(B, 1)
    We1a = We1[0:HID]
    We1c = We1[HID:2 * HID]
    w1last = We1[2 * HID].reshape(1, HID)
    pad = jnp.full((E_PAD - N_E,), PAD_IDX, jnp.int32)
    src_i = jnp.concatenate([edge_index[0].astype(jnp.int32), pad])
    dst_i = jnp.concatenate([edge_index[1].astype(jnp.int32), pad])
    keep = (dst_i >= N_P).astype(jnp.int32).reshape(NW, E_PER_TILE)
    pref_k = jnp.cumsum(keep, axis=1) - keep
    drop = 1 - keep
    pref_d = jnp.cumsum(drop, axis=1) - drop
    counts32 = keep.sum(axis=1)
    slotv = jnp.where(keep == 1, pref_k,
                      counts32[:, None] + pref_d).reshape(E_PAD)
    counts2d = jnp.broadcast_to(counts32[:, None], (NW, 16))
    Wv16 = jnp.concatenate([jnp.zeros((HID, 3), f32), Wv], axis=1)
    bv16 = jnp.concatenate([jnp.zeros((3,), f32), bv]).reshape(1, 16)

    ta, tc_t, tx, hlig = pl.pallas_call(
        _node_tables_kernel,
        out_shape=[
            jax.ShapeDtypeStruct((N_TAB, HID), f32),
            jax.ShapeDtypeStruct((N_TAB, HID), f32),
            jax.ShapeDtypeStruct((N_TAB, HID), f32),
            jax.ShapeDtypeStruct((N_L, HID), f32),
        ],
    )(protein_pos, protein_v, bp2, init_ligand_pos, lv2, bl2, tf,
      Wp, bp.reshape(1, HID - 1), Wl, bl.reshape(1, HID - 1),
      We1a, We1c, be1.reshape(1, HID))

    mesh = plsc.VectorSubcoreMesh(core_axis_name="c", subcore_axis_name="s")
    sc_params = pltpu.CompilerParams(use_tc_tiling_on_sc=True,
                                     needs_layout_passes=False)
    gather = functools.partial(
        pl.kernel, _sc_gather_kernel, mesh=mesh, compiler_params=sc_params,
        out_type=[
            jax.ShapeDtypeStruct((E_PAD, HID), f32),
            jax.ShapeDtypeStruct((E_PAD, HID), f32),
            jax.ShapeDtypeStruct((E_PAD, HID), f32),
            jax.ShapeDtypeStruct((E_PAD, HID), f32),
            jax.ShapeDtypeStruct((E_PAD,), jnp.int32),
        ],
        scratch_types=[
            pltpu.VMEM((E_PER_TILE,), jnp.int32),
            pltpu.VMEM((E_PER_TILE,), jnp.int32),
            pltpu.VMEM((E_PER_TILE,), jnp.int32),
            pltpu.VMEM((CBUF,), jnp.int32),
            pltpu.VMEM((CBUF,), jnp.int32),
            pltpu.VMEM((CHUNK,), jnp.int32),
            pltpu.VMEM((CHUNK,), jnp.int32),
            pltpu.VMEM((CHUNK, HID), f32),
            pltpu.VMEM((CHUNK, HID), f32),
            pltpu.VMEM((CHUNK, HID), f32),
            pltpu.VMEM((CHUNK, HID), f32),
            pltpu.VMEM((16,), jnp.int32),
            pltpu.SemaphoreType.DMA,
        ],
    )()
    a_rows, c_rows, xs_rows, xd_rows, dstc = gather(
        ta, tc_t, tx, src_i, dst_i, slotv, counts2d)

    def edge_blk(r, cdim):
        def im(t, j, counts):
            jm = jnp.maximum((counts[t] + EBLK - 1) // EBLK - 1, 0)
            return (t * NCHB + jnp.minimum(j, jm), 0)
        return pl.BlockSpec((r, cdim), im)

    rep = lambda r, cdim: pl.BlockSpec((r, cdim), lambda t, j, counts: (0, 0))
    m_rows, w_rows = pl.pallas_call(
        _edge_mlp_kernel,
        grid_spec=pltpu.PrefetchScalarGridSpec(
            num_scalar_prefetch=1,
            grid=(NW, NCHB),
            in_specs=[
                edge_blk(EBLK, HID), edge_blk(EBLK, HID),
                edge_blk(EBLK, HID), edge_blk(EBLK, HID),
                rep(1, HID), rep(HID, HID), rep(1, HID), rep(HID, HID),
                rep(1, HID), rep(1, HID), rep(1, 1),
            ],
            out_specs=[edge_blk(EBLK, HID), edge_blk(EBLK, HID)],
        ),
        out_shape=[
            jax.ShapeDtypeStruct((E_PAD, HID), f32),
            jax.ShapeDtypeStruct((E_PAD, HID), f32),
        ],
    )(counts32, a_rows, c_rows, xs_rows, xd_rows, w1last, We2,
      be2.reshape(1, HID), Wx1, bx1.reshape(1, HID),
      Wx2.reshape(1, HID), bx2.reshape(1, 1))

    scatter = functools.partial(
        pl.kernel, _sc_scatter_kernel, mesh=mesh, compiler_params=sc_params,
        out_type=[
            jax.ShapeDtypeStruct((NC, ACC_ROWS, HID), f32),
            jax.ShapeDtypeStruct((NC, ACC_ROWS, HID), f32),
        ],
        scratch_types=[
            pltpu.VMEM((CHUNK,), jnp.int32),
            pltpu.VMEM((CHUNK, HID), f32),
            pltpu.VMEM((CHUNK, HID), f32),
            pltpu.VMEM((CHUNK, HID), f32),
            pltpu.VMEM((CHUNK, HID), f32),
            pltpu.VMEM((16,), jnp.int32),
            pltpu.VMEM_SHARED((ACC_ROWS, HID), f32),
            pltpu.VMEM_SHARED((ACC_ROWS, HID), f32),
        ],
    )()
    pm, pw = scatter(m_rows, w_rows, dstc, counts2d)

    out = pl.pallas_call(
        _final_kernel,
        out_shape=jax.ShapeDtypeStruct((N_L, 16), f32),
    )(hlig, tx, pm, pw, Wn1, bn1.reshape(1, HID), Wn2, bn2.reshape(1, HID),
      Wv16, bv16)
    return out


# speedup vs baseline: 1.4623x; 1.3325x over previous
import functools

import jax
import jax.numpy as jnp
from jax import lax
from jax.experimental import pallas as pl
from jax.experimental.pallas import tpu as pltpu
from jax.experimental.pallas import tpu_sc as plsc

N_P, N_L, N_ALL, N_E = 8000, 2000, 10000, 160000
B, HID, PF, LF, T = 16, 128, 27, 13, 1000

N_TAB = 10240
PAD_IDX = 10112
PAD_LOCAL = PAD_IDX - N_P
NC, NS, NW = 2, 16, 32
CHUNK = 128
NCH = 40
E_PAD = NW * NCH * CHUNK
E_PER_TILE = NCH * CHUNK
G16 = E_PER_TILE // 16
CBUF = E_PER_TILE
ACC_ROWS = 2304
LIG_ROWS = N_TAB - N_P
ROWS_PER_TILE = LIG_ROWS // NS
EBLK = 512
NCHB = E_PER_TILE // EBLK


def _node_tables_kernel(ppos_ref, pv_ref, bp_ref, lpos_ref, lv_ref, bl_ref,
                        tf_ref, Wp_ref, bpv_ref, Wl_ref, blv_ref,
                        We1a_ref, We1c_ref, be1_ref,
                        ta_ref, tc_ref, tx_ref, hlig_ref):
    bp = bp_ref[...]
    ohp = (bp == lax.broadcasted_iota(jnp.int32, (N_P, B), 1)).astype(jnp.float32)
    ppos = ppos_ref[...]
    seg = lax.dot_general(ohp, ppos, (((0,), (0,)), ((), ())))
    cnt = lax.dot_general(ohp, jnp.ones((N_P, 1), jnp.float32),
                          (((0,), (0,)), ((), ())))
    offset = seg / jnp.maximum(cnt, 1.0)
    bl = bl_ref[...]
    ohl = (bl == lax.broadcasted_iota(jnp.int32, (N_L, B), 1)).astype(jnp.float32)
    pc = ppos - ohp @ offset
    lc = lpos_ref[...] - ohl @ offset

    h_p = pv_ref[...] @ Wp_ref[...] + bpv_ref[...]
    onehot = (lv_ref[...] == lax.broadcasted_iota(jnp.int32, (N_L, LF), 1)
              ).astype(jnp.float32)
    tfeat = ohl @ tf_ref[...]
    lig_feat = jnp.concatenate([onehot, tfeat], axis=1)
    h_l = lig_feat @ Wl_ref[...] + blv_ref[...]

    We1a = We1a_ref[...]
    We1c = We1c_ref[...]
    a_p = h_p @ We1a[:127] + be1_ref[...]
    c_p = h_p @ We1c[:127]
    a_l = h_l @ We1a[:127] + We1a[127:128] + be1_ref[...]
    c_l = h_l @ We1c[:127] + We1c[127:128]

    ta_ref[0:N_P, :] = a_p
    ta_ref[N_P:N_ALL, :] = a_l
    ta_ref[N_ALL:N_TAB, :] = jnp.zeros((N_TAB - N_ALL, HID), jnp.float32)
    tc_ref[0:N_P, :] = c_p
    tc_ref[N_P:N_ALL, :] = c_l
    tc_ref[N_ALL:N_TAB, :] = jnp.zeros((N_TAB - N_ALL, HID), jnp.float32)
    tx_ref[0:N_P, :] = jnp.concatenate(
        [pc, jnp.zeros((N_P, 13), jnp.float32)], axis=1)
    tx_ref[N_P:N_ALL, :] = jnp.concatenate(
        [lc, jnp.zeros((N_L, 13), jnp.float32)], axis=1)
    tx_ref[N_ALL:N_TAB, :] = jnp.zeros((N_TAB - N_ALL, 16), jnp.float32)
    hlig_ref[...] = jnp.concatenate(
        [h_l, jnp.ones((N_L, 1), jnp.float32)], axis=1)


def _sc_gather_kernel(ta, tc_t, tx, src_i, dst_i, slotv, counts,
                      a_out, c_out, xs_out, xd_out, dstc_out,
                      sraw, draw, slotbuf, scmp, dcmp, idxs_v, idxd_v,
                      abuf, cbuf, xsbuf, xdbuf, cnt_v, sem):
    wid = lax.axis_index("s") * NC + lax.axis_index("c")
    base = wid * E_PER_TILE
    pltpu.sync_copy(src_i.at[pl.ds(base, E_PER_TILE)], sraw)
    pltpu.sync_copy(dst_i.at[pl.ds(base, E_PER_TILE)], draw)
    pltpu.sync_copy(slotv.at[pl.ds(base, E_PER_TILE)], slotbuf)
    pltpu.sync_copy(counts.at[wid], cnt_v)
    cnt = jnp.max(cnt_v[...])

    def compact(i, carry):
        dv = draw[pl.ds(i * 16, 16)]
        sv = sraw[pl.ds(i * 16, 16)]
        pv = slotbuf[pl.ds(i * 16, 16)]
        dl = dv - N_P
        neg = lax.shift_right_logical(dl, 31)
        dval = dl + neg * (PAD_LOCAL - dl)
        sval = sv + neg * (PAD_IDX - sv)
        plsc.store_scatter(dcmp, [pv], dval)
        plsc.store_scatter(scmp, [pv], sval)
        return carry

    lax.fori_loop(0, G16, compact, 0)
    pltpu.sync_copy(dcmp.at[pl.ds(0, E_PER_TILE)],
                    dstc_out.at[pl.ds(base, E_PER_TILE)])

    def body(i, carry):
        @pl.when(i * CHUNK < cnt)
        def _():
            off = i * CHUNK
            for g in range(CHUNK // 16):
                idxs_v[pl.ds(g * 16, 16)] = scmp[pl.ds(off + g * 16, 16)]
                idxd_v[pl.ds(g * 16, 16)] = dcmp[pl.ds(off + g * 16, 16)] + N_P
            d_a = pltpu.async_copy(ta.at[idxs_v], abuf, sem)
            d_c = pltpu.async_copy(tc_t.at[idxd_v], cbuf, sem)
            d_xs = pltpu.async_copy(tx.at[idxs_v], xsbuf, sem)
            d_xd = pltpu.async_copy(tx.at[idxd_v], xdbuf, sem)
            d_a.wait(); d_c.wait(); d_xs.wait(); d_xd.wait()
            pltpu.sync_copy(abuf, a_out.at[pl.ds(base + off, CHUNK)])
            pltpu.sync_copy(cbuf, c_out.at[pl.ds(base + off, CHUNK)])
            pltpu.sync_copy(xsbuf, xs_out.at[pl.ds(base + off, CHUNK)])
            pltpu.sync_copy(xdbuf, xd_out.at[pl.ds(base + off, CHUNK)])
        return carry

    lax.fori_loop(0, NCH, body, 0)


def _edge_mlp_kernel(counts_ref, a_ref, c_ref, xs_ref, xd_ref,
                     w1l_ref, We2_ref, be2_ref, Wx1_ref, bx1_ref,
                     wx2_ref, bx2_ref, m_ref, w_ref):
    t = pl.program_id(0)
    j = pl.program_id(1)

    @pl.when(j * EBLK < counts_ref[t])
    def _():
        pre = a_ref[...] + c_ref[...]
        rel16 = xd_ref[...] - xs_ref[...]
        d2 = jnp.sum(rel16 * rel16, axis=1, keepdims=True)
        m1 = jax.nn.silu(pre + d2 * w1l_ref[...])
        m2 = jax.nn.silu(m1 @ We2_ref[...] + be2_ref[...])
        u = jax.nn.silu(m2 @ Wx1_ref[...] + bx1_ref[...])
        xw = jnp.sum(u * wx2_ref[...], axis=1, keepdims=True) + bx2_ref[0, 0]
        m_ref[...] = m2
        w_ref[...] = rel16 * jnp.tanh(xw)


def _sc_scatter_kernel(ym, yw, dstc, counts, aggm_out, aggw_out,
                       idx_v, mbuf, wbuf, zm, zw, cnt_v,
                       aggm_acc, aggw_acc):
    c = lax.axis_index("c")
    s = lax.axis_index("s")
    wid = s * NC + c
    pltpu.sync_copy(counts.at[wid], cnt_v)
    cnt = jnp.max(cnt_v[...])

    def zrow(i, carry):
        for g in range(HID // 16):
            zm[i, pl.ds(g * 16, 16)] = jnp.zeros((16,), jnp.float32)
        zw[i, pl.ds(0, 16)] = jnp.zeros((16,), jnp.float32)
        return carry

    lax.fori_loop(0, CHUNK, zrow, 0)
    zbase = s * (ACC_ROWS // NS)
    pltpu.sync_copy(zm, aggm_acc.at[pl.ds(zbase, CHUNK)])
    pltpu.sync_copy(zw, aggw_acc.at[pl.ds(zbase, CHUNK)])
    pltpu.sync_copy(zm.at[pl.ds(0, 16)],
                    aggm_acc.at[pl.ds(zbase + CHUNK, 16)])
    pltpu.sync_copy(zw.at[pl.ds(0, 16)],
                    aggw_acc.at[pl.ds(zbase + CHUNK, 16)])
    plsc.subcore_barrier()

    def body(i, carry):
        @pl.when(i * CHUNK < cnt)
        def _():
            base = wid * E_PER_TILE + i * CHUNK
            pltpu.sync_copy(dstc.at[pl.ds(base, CHUNK)], idx_v)
            pltpu.sync_copy(ym.at[pl.ds(base, CHUNK)], mbuf)
            pltpu.sync_copy(yw.at[pl.ds(base, CHUNK)], wbuf)
            pltpu.sync_copy(mbuf, aggm_acc.at[idx_v], add=True)
            pltpu.sync_copy(wbuf, aggw_acc.at[idx_v], add=True)
        return carry

    lax.fori_loop(0, NCH, body, 0)
    plsc.subcore_barrier()
    rbase = s * ROWS_PER_TILE
    pltpu.sync_copy(aggm_acc.at[pl.ds(rbase, ROWS_PER_TILE)],
                    aggm_out.at[c, pl.ds(rbase, ROWS_PER_TILE)])
    pltpu.sync_copy(aggw_acc.at[pl.ds(rbase, ROWS_PER_TILE)],
                    aggw_out.at[c, pl.ds(rbase, ROWS_PER_TILE)])


def _final_kernel(hlig_ref, tx_ref, pm_ref, pw_ref,
                  Wn1_ref, bn1_ref, Wn2_ref, bn2_ref, Wv_ref, bv_ref,
                  out_ref):
    h = hlig_ref[...]
    agg = pm_ref[0, 0:N_L, :] + pm_ref[1, 0:N_L, :]
    dx16 = pw_ref[0, 0:N_L, :] + pw_ref[1, 0:N_L, :]
    x16 = tx_ref[N_P:N_ALL, :]
    cat = jnp.concatenate([h, agg], axis=1)
    hn = jax.nn.silu(cat @ Wn1_ref[...] + bn1_ref[...])
    hnew = h + hn @ Wn2_ref[...] + bn2_ref[...]
    out_ref[...] = x16 + dx16 + hnew @ Wv_ref[...] + bv_ref[...]


def kernel(protein_pos, protein_v, batch_protein, init_ligand_pos,
           init_ligand_v, batch_ligand, time_step, edge_index,
           Wp, bp, Wl, bl, We1, be1, We2, be2, Wn1, bn1, Wn2, bn2,
           Wx1, bx1, Wx2, bx2, Wv, bv):
    f32 = jnp.float32

    bp2 = batch_protein.astype(jnp.int32).reshape(N_P, 1)
    bl2 = batch_ligand.astype(jnp.int32).reshape(N_L, 1)
    lv2 = init_ligand_v.astype(jnp.int32).reshape(N_L, 1)
    tf = (time_step.astype(f32) / T).reshape(B, 1)
    We1a = We1[0:HID]
    We1c = We1[HID:2 * HID]
    w1last = We1[2 * HID].reshape(1, HID)
    pad = jnp.full((E_PAD - N_E,), PAD_IDX, jnp.int32)
    src_i = jnp.concatenate([edge_index[0].astype(jnp.int32), pad])
    dst_i = jnp.concatenate([edge_index[1].astype(jnp.int32), pad])
    keep = (dst_i >= N_P).astype(jnp.int32).reshape(NW, E_PER_TILE)
    pref_k = jnp.cumsum(keep, axis=1) - keep
    drop = 1 - keep
    pref_d = jnp.cumsum(drop, axis=1) - drop
    counts32 = keep.sum(axis=1)
    slotv = jnp.where(keep == 1, pref_k,
                      counts32[:, None] + pref_d).reshape(E_PAD)
    counts2d = jnp.broadcast_to(counts32[:, None], (NW, 16))
    Wv16 = jnp.concatenate([jnp.zeros((HID, 3), f32), Wv], axis=1)
    bv16 = jnp.concatenate([jnp.zeros((3,), f32), bv]).reshape(1, 16)

    ta, tc_t, tx, hlig = pl.pallas_call(
        _node_tables_kernel,
        out_shape=[
            jax.ShapeDtypeStruct((N_TAB, HID), f32),
            jax.ShapeDtypeStruct((N_TAB, HID), f32),
            jax.ShapeDtypeStruct((N_TAB, 16), f32),
            jax.ShapeDtypeStruct((N_L, HID), f32),
        ],
    )(protein_pos, protein_v, bp2, init_ligand_pos, lv2, bl2, tf,
      Wp, bp.reshape(1, HID - 1), Wl, bl.reshape(1, HID - 1),
      We1a, We1c, be1.reshape(1, HID))

    mesh = plsc.VectorSubcoreMesh(core_axis_name="c", subcore_axis_name="s")
    sc_params = pltpu.CompilerParams(use_tc_tiling_on_sc=False,
                                     needs_layout_passes=False)
    gather = functools.partial(
        pl.kernel, _sc_gather_kernel, mesh=mesh, compiler_params=sc_params,
        out_type=[
            jax.ShapeDtypeStruct((E_PAD, HID), f32),
            jax.ShapeDtypeStruct((E_PAD, HID), f32),
            jax.ShapeDtypeStruct((E_PAD, 16), f32),
            jax.ShapeDtypeStruct((E_PAD, 16), f32),
            jax.ShapeDtypeStruct((E_PAD,), jnp.int32),
        ],
        scratch_types=[
            pltpu.VMEM((E_PER_TILE,), jnp.int32),
            pltpu.VMEM((E_PER_TILE,), jnp.int32),
            pltpu.VMEM((E_PER_TILE,), jnp.int32),
            pltpu.VMEM((CBUF,), jnp.int32),
            pltpu.VMEM((CBUF,), jnp.int32),
            pltpu.VMEM((CHUNK,), jnp.int32),
            pltpu.VMEM((CHUNK,), jnp.int32),
            pltpu.VMEM((CHUNK, HID), f32),
            pltpu.VMEM((CHUNK, HID), f32),
            pltpu.VMEM((CHUNK, 16), f32),
            pltpu.VMEM((CHUNK, 16), f32),
            pltpu.VMEM((16,), jnp.int32),
            pltpu.SemaphoreType.DMA,
        ],
    )()
    a_rows, c_rows, xs_rows, xd_rows, dstc = gather(
        ta, tc_t, tx, src_i, dst_i, slotv, counts2d)

    def edge_blk(r, cdim):
        def im(t, j, counts):
            jm = jnp.maximum((counts[t] + EBLK - 1) // EBLK - 1, 0)
            return (t * NCHB + jnp.minimum(j, jm), 0)
        return pl.BlockSpec((r, cdim), im)

    rep = lambda r, cdim: pl.BlockSpec((r, cdim), lambda t, j, counts: (0, 0))
    m_rows, w_rows = pl.pallas_call(
        _edge_mlp_kernel,
        grid_spec=pltpu.PrefetchScalarGridSpec(
            num_scalar_prefetch=1,
            grid=(NW, NCHB),
            in_specs=[
                edge_blk(EBLK, HID), edge_blk(EBLK, HID),
                edge_blk(EBLK, 16), edge_blk(EBLK, 16),
                rep(1, HID), rep(HID, HID), rep(1, HID), rep(HID, HID),
                rep(1, HID), rep(1, HID), rep(1, 1),
            ],
            out_specs=[edge_blk(EBLK, HID), edge_blk(EBLK, 16)],
        ),
        out_shape=[
            jax.ShapeDtypeStruct((E_PAD, HID), f32),
            jax.ShapeDtypeStruct((E_PAD, 16), f32),
        ],
    )(counts32, a_rows, c_rows, xs_rows, xd_rows, w1last, We2,
      be2.reshape(1, HID), Wx1, bx1.reshape(1, HID),
      Wx2.reshape(1, HID), bx2.reshape(1, 1))

    scatter = functools.partial(
        pl.kernel, _sc_scatter_kernel, mesh=mesh, compiler_params=sc_params,
        out_type=[
            jax.ShapeDtypeStruct((NC, LIG_ROWS, HID), f32),
            jax.ShapeDtypeStruct((NC, LIG_ROWS, 16), f32),
        ],
        scratch_types=[
            pltpu.VMEM((CHUNK,), jnp.int32),
            pltpu.VMEM((CHUNK, HID), f32),
            pltpu.VMEM((CHUNK, 16), f32),
            pltpu.VMEM((CHUNK, HID), f32),
            pltpu.VMEM((CHUNK, 16), f32),
            pltpu.VMEM((16,), jnp.int32),
            pltpu.VMEM_SHARED((ACC_ROWS, HID), f32),
            pltpu.VMEM_SHARED((ACC_ROWS, 16), f32),
        ],
    )()
    pm, pw = scatter(m_rows, w_rows, dstc, counts2d)

    out = pl.pallas_call(
        _final_kernel,
        out_shape=jax.ShapeDtypeStruct((N_L, 16), f32),
    )(hlig, tx, pm, pw, Wn1, bn1.reshape(1, HID), Wn2, bn2.reshape(1, HID),
      Wv16, bv16)
    return out
